# baseline (speedup 1.0000x reference)
import functools

import jax
import jax.numpy as jnp
from jax import lax
from jax.experimental import pallas as pl
from jax.experimental.pallas import tpu as pltpu
from jax.experimental.pallas import tpu_sc as plsc

_L = 16
_NW = 32


def _sqrt16(x):
    xc = jnp.maximum(x, jnp.float32(1e-30))
    i = lax.bitcast_convert_type(xc, jnp.int32)
    y = lax.bitcast_convert_type(
        jnp.int32(0x5F3759DF) - lax.shift_right_arithmetic(i, 1), jnp.float32)
    xh = xc * jnp.float32(0.5)
    for _ in range(3):
        y = y * (jnp.float32(1.5) - xh * y * y)
    return xc * y


def make_sc_chamfer(scb: int, n: int):
    g = _NW // scb
    qpw = n // g
    nkv = n // _L
    qv_n = qpw // _L

    mesh = plsc.VectorSubcoreMesh(core_axis_name="c", subcore_axis_name="s")

    @functools.partial(
        pl.kernel,
        mesh=mesh,
        out_type=[
            jax.ShapeDtypeStruct((_NW, _L), jnp.float32),
            jax.ShapeDtypeStruct((scb, _L), jnp.float32),
        ],
        scratch_types=[
            pltpu.VMEM((3, n), jnp.float32),
            pltpu.VMEM((3 * qpw,), jnp.float32),
            pltpu.VMEM((3 * qpw,), jnp.float32),
            pltpu.VMEM((n,), jnp.float32),
            pltpu.VMEM((_L,), jnp.float32),
            pltpu.VMEM((_L,), jnp.float32),
            pltpu.VMEM((g, n), jnp.float32),
            pltpu.VMEM_SHARED((16, n), jnp.float32),
        ],
    )
    def sc_chamfer(qpc1_hbm, qflow_hbm, pc2t_hbm, rowp_hbm, colp_hbm,
                   keys_v, qs_v, qf_v, cmin_v, racc_v, cacc_v,
                   tmp_v, shared):
        c = lax.axis_index("c")
        s = lax.axis_index("s")
        wid = c * 16 + s
        b = wid // g

        pltpu.sync_copy(pc2t_hbm.at[b], keys_v)
        pltpu.sync_copy(qpc1_hbm.at[wid], qs_v)
        pltpu.sync_copy(qflow_hbm.at[wid], qf_v)

        def add_body(i, _):
            sl = pl.ds(i * _L, _L)
            qs_v[sl] = qs_v[sl] + qf_v[sl]
            return 0
        lax.fori_loop(0, 3 * qpw // _L, add_body, 0)

        inf16 = jnp.full((_L,), jnp.inf, jnp.float32)
        lane_iota = lax.iota(jnp.int32, _L)

        def lanemin(x):
            for k in (8, 4, 2, 1):
                perm = jnp.bitwise_xor(lane_iota, jnp.int32(k))
                x = jnp.minimum(x, x.at[perm].get(mode="promise_in_bounds"))
            return x

        def splat(vec, l):
            idx = jnp.full((_L,), l, jnp.int32)
            return vec.at[idx].get(mode="promise_in_bounds")

        def init_body(i, _):
            cmin_v[pl.ds(i * _L, _L)] = inf16
            return 0
        lax.fori_loop(0, nkv, init_body, 0)

        nkb = nkv // 4

        def qv_body(qv, racc):
            qvx = qs_v[pl.ds(qv * _L, _L)]
            qvy = qs_v[pl.ds(qpw + qv * _L, _L)]
            qvz = qs_v[pl.ds(2 * qpw + qv * _L, _L)]

            def make_kb_body(l0):
                def kb_body(kb, rms):
                    base = kb * 4 * _L
                    kx = [keys_v[0, pl.ds(base + v * _L, _L)] for v in range(4)]
                    ky = [keys_v[1, pl.ds(base + v * _L, _L)] for v in range(4)]
                    kz = [keys_v[2, pl.ds(base + v * _L, _L)] for v in range(4)]
                    cm = [cmin_v[pl.ds(base + v * _L, _L)] for v in range(4)]
                    new_rms = []
                    for l in range(l0, l0 + 8):
                        qx = splat(qvx, l)
                        qy = splat(qvy, l)
                        qz = splat(qvz, l)
                        rm = rms[l - l0]
                        for v in range(4):
                            dx = qx - kx[v]
                            t = dx * dx
                            dy = qy - ky[v]
                            t = dy * dy + t
                            dz = qz - kz[v]
                            t = dz * dz + t
                            rm = jnp.minimum(rm, t)
                            cm[v] = jnp.minimum(cm[v], t)
                        new_rms.append(rm)
                    for v in range(4):
                        cmin_v[pl.ds(base + v * _L, _L)] = cm[v]
                    return tuple(new_rms)
                return kb_body

            for l0 in (0, 8):
                rms = lax.fori_loop(0, nkb, make_kb_body(l0), (inf16,) * 8)
                for l in range(8):
                    racc = racc + _sqrt16(lanemin(rms[l]))
            return racc

        racc = lax.fori_loop(0, qv_n, qv_body, jnp.zeros((_L,), jnp.float32))
        racc_v[...] = racc
        pltpu.sync_copy(racc_v, rowp_hbm.at[wid])

        pltpu.sync_copy(cmin_v, shared.at[s])
        plsc.subcore_barrier()

        @pl.when(s % g == 0)
        def _():
            for j in range(g):
                pltpu.sync_copy(shared.at[s + j], tmp_v.at[j])

            def c_body(i, acc):
                sl = pl.ds(i * _L, _L)
                m = tmp_v[0, sl]
                for j in range(1, g):
                    m = jnp.minimum(m, tmp_v[j, sl])
                return acc + _sqrt16(m)

            cacc = lax.fori_loop(0, nkv, c_body, jnp.zeros((_L,), jnp.float32))
            cacc_v[...] = cacc
            pltpu.sync_copy(cacc_v, colp_hbm.at[b])

    return sc_chamfer


_TM = 256


def _tc_chamfer_body(pc1_ref, flow_ref, pc2t_ref, out_ref):
    n = pc1_ref.shape[1]
    p1 = pc1_ref[0] + flow_ref[0]
    c2x = pc2t_ref[0, 0:1, :]
    c2y = pc2t_ref[0, 1:2, :]
    c2z = pc2t_ref[0, 2:3, :]

    row_sum = jnp.zeros((), dtype=jnp.float32)
    col_min = jnp.full((1, n), jnp.inf, dtype=jnp.float32)
    for i in range(n // _TM):
        sl = p1[i * _TM:(i + 1) * _TM]
        dx = sl[:, 0:1] - c2x
        d = dx * dx
        dy = sl[:, 1:2] - c2y
        d += dy * dy
        dz = sl[:, 2:3] - c2z
        d += dz * dz
        row_min = jnp.min(d, axis=1)
        row_sum += jnp.sum(jnp.sqrt(row_min))
        col_min = jnp.minimum(col_min, jnp.min(d, axis=0, keepdims=True))

    total = row_sum + jnp.sum(jnp.sqrt(col_min))
    out_ref[0] = total.reshape(1, 1)


def _tc_chamfer(pc1, pc2t, flow):
    b, n, _ = pc1.shape
    return pl.pallas_call(
        _tc_chamfer_body,
        grid=(b,),
        in_specs=[
            pl.BlockSpec((1, n, 3), lambda i: (i, 0, 0)),
            pl.BlockSpec((1, n, 3), lambda i: (i, 0, 0)),
            pl.BlockSpec((1, 3, n), lambda i: (i, 0, 0)),
        ],
        out_specs=pl.BlockSpec((1, 1, 1), lambda i: (i, 0, 0)),
        out_shape=jax.ShapeDtypeStruct((b, 1, 1), jnp.float32),
        compiler_params=pltpu.CompilerParams(
            dimension_semantics=("parallel",),
        ),
    )(pc1, flow, pc2t)


_SCB = 2


def kernel(pc1, pc2, flow):
    b, n, _ = pc1.shape
    tcb = b - _SCB
    g = _NW // _SCB
    qpw = n // g
    pc2t = jnp.transpose(pc2, (0, 2, 1))

    pc1t = jnp.transpose(pc1[tcb:], (0, 2, 1))
    flowt = jnp.transpose(flow[tcb:], (0, 2, 1))
    qpc1 = pc1t.reshape(_SCB, 3, g, qpw).transpose(0, 2, 1, 3).reshape(
        _NW, 3 * qpw)
    qflow = flowt.reshape(_SCB, 3, g, qpw).transpose(0, 2, 1, 3).reshape(
        _NW, 3 * qpw)
    rowp, colp = make_sc_chamfer(_SCB, n)(qpc1, qflow, pc2t[tcb:])

    tc_part = _tc_chamfer(pc1[:tcb], pc2t[:tcb], flow[:tcb])

    return (jnp.sum(tc_part) + jnp.sum(rowp) / _L + jnp.sum(colp)) / (b * n)


# speedup vs baseline: 1.1018x; 1.1018x over previous
import functools

import jax
import jax.numpy as jnp
from jax import lax
from jax.experimental import pallas as pl
from jax.experimental.pallas import tpu as pltpu
from jax.experimental.pallas import tpu_sc as plsc

_L = 16
_NW = 32


def _sqrt16(x):
    xc = jnp.maximum(x, jnp.float32(1e-30))
    i = lax.bitcast_convert_type(xc, jnp.int32)
    y = lax.bitcast_convert_type(
        jnp.int32(0x5F3759DF) - lax.shift_right_arithmetic(i, 1), jnp.float32)
    xh = xc * jnp.float32(0.5)
    for _ in range(3):
        y = y * (jnp.float32(1.5) - xh * y * y)
    return xc * y


def make_sc_chamfer(scb: int, n: int):
    g = _NW // scb
    qpw = n // g
    nkv = n // _L
    qv_n = qpw // _L

    mesh = plsc.VectorSubcoreMesh(core_axis_name="c", subcore_axis_name="s")

    @functools.partial(
        pl.kernel,
        mesh=mesh,
        out_type=[
            jax.ShapeDtypeStruct((_NW, _L), jnp.float32),
            jax.ShapeDtypeStruct((scb, _L), jnp.float32),
        ],
        scratch_types=[
            pltpu.VMEM((3, n), jnp.float32),
            pltpu.VMEM((3 * qpw,), jnp.float32),
            pltpu.VMEM((3 * qpw,), jnp.float32),
            pltpu.VMEM((n,), jnp.float32),
            pltpu.VMEM((_L,), jnp.float32),
            pltpu.VMEM((_L,), jnp.float32),
            pltpu.VMEM((g, n), jnp.float32),
            pltpu.VMEM_SHARED((16, n), jnp.float32),
        ],
    )
    def sc_chamfer(qpc1_hbm, qflow_hbm, pc2t_hbm, rowp_hbm, colp_hbm,
                   keys_v, qs_v, qf_v, cmin_v, racc_v, cacc_v,
                   tmp_v, shared):
        c = lax.axis_index("c")
        s = lax.axis_index("s")
        wid = c * 16 + s
        b = wid // g

        pltpu.sync_copy(pc2t_hbm.at[b], keys_v)
        pltpu.sync_copy(qpc1_hbm.at[wid], qs_v)
        pltpu.sync_copy(qflow_hbm.at[wid], qf_v)

        def add_body(i, _):
            sl = pl.ds(i * _L, _L)
            qs_v[sl] = qs_v[sl] + qf_v[sl]
            return 0
        lax.fori_loop(0, 3 * qpw // _L, add_body, 0)

        inf16 = jnp.full((_L,), jnp.inf, jnp.float32)
        lane_iota = lax.iota(jnp.int32, _L)

        def lanemin(x):
            for k in (8, 4, 2, 1):
                perm = jnp.bitwise_xor(lane_iota, jnp.int32(k))
                x = jnp.minimum(x, x.at[perm].get(mode="promise_in_bounds"))
            return x

        def splat(vec, l):
            idx = jnp.full((_L,), l, jnp.int32)
            return vec.at[idx].get(mode="promise_in_bounds")

        def init_body(i, _):
            cmin_v[pl.ds(i * _L, _L)] = inf16
            return 0
        lax.fori_loop(0, nkv, init_body, 0)

        nkb = nkv // 4

        def qv_body(qv, racc):
            qvx = qs_v[pl.ds(qv * _L, _L)]
            qvy = qs_v[pl.ds(qpw + qv * _L, _L)]
            qvz = qs_v[pl.ds(2 * qpw + qv * _L, _L)]

            def make_kb_body(l0):
                def kb_body(kb, rms):
                    base = kb * 4 * _L
                    kx = [keys_v[0, pl.ds(base + v * _L, _L)] for v in range(4)]
                    ky = [keys_v[1, pl.ds(base + v * _L, _L)] for v in range(4)]
                    kz = [keys_v[2, pl.ds(base + v * _L, _L)] for v in range(4)]
                    cm = [cmin_v[pl.ds(base + v * _L, _L)] for v in range(4)]
                    new_rms = []
                    for l in range(l0, l0 + 8):
                        qx = splat(qvx, l)
                        qy = splat(qvy, l)
                        qz = splat(qvz, l)
                        rm = rms[l - l0]
                        for v in range(4):
                            dx = qx - kx[v]
                            t = dx * dx
                            dy = qy - ky[v]
                            t = dy * dy + t
                            dz = qz - kz[v]
                            t = dz * dz + t
                            rm = jnp.minimum(rm, t)
                            cm[v] = jnp.minimum(cm[v], t)
                        new_rms.append(rm)
                    for v in range(4):
                        cmin_v[pl.ds(base + v * _L, _L)] = cm[v]
                    return tuple(new_rms)
                return kb_body

            for l0 in (0, 8):
                rms = lax.fori_loop(0, nkb, make_kb_body(l0), (inf16,) * 8)
                for l in range(8):
                    racc = racc + _sqrt16(lanemin(rms[l]))
            return racc

        racc = lax.fori_loop(0, qv_n, qv_body, jnp.zeros((_L,), jnp.float32))
        racc_v[...] = racc
        pltpu.sync_copy(racc_v, rowp_hbm.at[wid])

        pltpu.sync_copy(cmin_v, shared.at[s])
        plsc.subcore_barrier()

        @pl.when(s % g == 0)
        def _():
            for j in range(g):
                pltpu.sync_copy(shared.at[s + j], tmp_v.at[j])

            def c_body(i, acc):
                sl = pl.ds(i * _L, _L)
                m = tmp_v[0, sl]
                for j in range(1, g):
                    m = jnp.minimum(m, tmp_v[j, sl])
                return acc + _sqrt16(m)

            cacc = lax.fori_loop(0, nkv, c_body, jnp.zeros((_L,), jnp.float32))
            cacc_v[...] = cacc
            pltpu.sync_copy(cacc_v, colp_hbm.at[b])

    return sc_chamfer


_TM = 256


def _tc_chamfer_body(pc1t_ref, flowt_ref, pc2t_ref, out_ref):
    n = pc1t_ref.shape[2]
    p1t = pc1t_ref[0] + flowt_ref[0]
    p1 = jnp.transpose(p1t)
    c2x = pc2t_ref[0, 0:1, :]
    c2y = pc2t_ref[0, 1:2, :]
    c2z = pc2t_ref[0, 2:3, :]

    row_sum = jnp.zeros((), dtype=jnp.float32)
    col_min = jnp.full((1, n), jnp.inf, dtype=jnp.float32)
    for i in range(n // _TM):
        sl = p1[i * _TM:(i + 1) * _TM]
        dx = sl[:, 0:1] - c2x
        d = dx * dx
        dy = sl[:, 1:2] - c2y
        d += dy * dy
        dz = sl[:, 2:3] - c2z
        d += dz * dz
        row_min = jnp.min(d, axis=1)
        row_sum += jnp.sum(jnp.sqrt(row_min))
        col_min = jnp.minimum(col_min, jnp.min(d, axis=0, keepdims=True))

    total = row_sum + jnp.sum(jnp.sqrt(col_min))
    out_ref[0] = total.reshape(1, 1)


def _tc_chamfer(pc1t, pc2t, flowt):
    b, _, n = pc1t.shape
    return pl.pallas_call(
        _tc_chamfer_body,
        grid=(b,),
        in_specs=[
            pl.BlockSpec((1, 3, n), lambda i: (i, 0, 0)),
            pl.BlockSpec((1, 3, n), lambda i: (i, 0, 0)),
            pl.BlockSpec((1, 3, n), lambda i: (i, 0, 0)),
        ],
        out_specs=pl.BlockSpec((1, 1, 1), lambda i: (i, 0, 0)),
        out_shape=jax.ShapeDtypeStruct((b, 1, 1), jnp.float32),
        compiler_params=pltpu.CompilerParams(
            dimension_semantics=("parallel",),
        ),
    )(pc1t, flowt, pc2t)


_SCB = 2


def kernel(pc1, pc2, flow):
    b, n, _ = pc1.shape
    tcb = b - _SCB
    g = _NW // _SCB
    qpw = n // g
    pc2t = jnp.transpose(pc2, (0, 2, 1))
    pc1t = jnp.transpose(pc1, (0, 2, 1))
    flowt = jnp.transpose(flow, (0, 2, 1))

    qpc1 = pc1t[tcb:].reshape(_SCB, 3, g, qpw).transpose(0, 2, 1, 3).reshape(
        _NW, 3 * qpw)
    qflow = flowt[tcb:].reshape(_SCB, 3, g, qpw).transpose(0, 2, 1, 3).reshape(
        _NW, 3 * qpw)
    rowp, colp = make_sc_chamfer(_SCB, n)(qpc1, qflow, pc2t[tcb:])

    tc_part = _tc_chamfer(pc1t[:tcb], pc2t[:tcb], flowt[:tcb])

    return (jnp.sum(tc_part) + jnp.sum(rowp) / _L + jnp.sum(colp)) / (b * n)


# speedup vs baseline: 1.2308x; 1.1172x over previous
import functools

import jax
import jax.numpy as jnp
from jax import lax
from jax.experimental import pallas as pl
from jax.experimental.pallas import tpu as pltpu
from jax.experimental.pallas import tpu_sc as plsc

_L = 16
_NW = 32


def _sqrt16(x):
    xc = jnp.maximum(x, jnp.float32(1e-30))
    i = lax.bitcast_convert_type(xc, jnp.int32)
    y = lax.bitcast_convert_type(
        jnp.int32(0x5F3759DF) - lax.shift_right_arithmetic(i, 1), jnp.float32)
    xh = xc * jnp.float32(0.5)
    for _ in range(3):
        y = y * (jnp.float32(1.5) - xh * y * y)
    return xc * y


def make_sc_chamfer(scb: int, n: int):
    g = _NW // scb
    qpw = n // g
    nkv = n // _L
    qv_n = qpw // _L

    mesh = plsc.VectorSubcoreMesh(core_axis_name="c", subcore_axis_name="s")

    @functools.partial(
        pl.kernel,
        mesh=mesh,
        out_type=[
            jax.ShapeDtypeStruct((_NW, _L), jnp.float32),
            jax.ShapeDtypeStruct((scb, _L), jnp.float32),
        ],
        scratch_types=[
            pltpu.VMEM((3, n), jnp.float32),
            pltpu.VMEM((3 * qpw,), jnp.float32),
            pltpu.VMEM((3 * qpw,), jnp.float32),
            pltpu.VMEM((n,), jnp.float32),
            pltpu.VMEM((_L,), jnp.float32),
            pltpu.VMEM((_L,), jnp.float32),
            pltpu.VMEM((g, n), jnp.float32),
            pltpu.VMEM_SHARED((16, n), jnp.float32),
        ],
    )
    def sc_chamfer(qpc1_hbm, qflow_hbm, pc2t_hbm, rowp_hbm, colp_hbm,
                   keys_v, qs_v, qf_v, cmin_v, racc_v, cacc_v,
                   tmp_v, shared):
        c = lax.axis_index("c")
        s = lax.axis_index("s")
        wid = c * 16 + s
        b = wid // g

        pltpu.sync_copy(pc2t_hbm.at[b], keys_v)
        pltpu.sync_copy(qpc1_hbm.at[wid], qs_v)
        pltpu.sync_copy(qflow_hbm.at[wid], qf_v)

        def add_body(i, _):
            sl = pl.ds(i * _L, _L)
            qs_v[sl] = qs_v[sl] + qf_v[sl]
            return 0
        lax.fori_loop(0, 3 * qpw // _L, add_body, 0)

        inf16 = jnp.full((_L,), jnp.inf, jnp.float32)
        lane_iota = lax.iota(jnp.int32, _L)

        def lanemin(x):
            for k in (8, 4, 2, 1):
                perm = jnp.bitwise_xor(lane_iota, jnp.int32(k))
                x = jnp.minimum(x, x.at[perm].get(mode="promise_in_bounds"))
            return x

        def splat(vec, l):
            idx = jnp.full((_L,), l, jnp.int32)
            return vec.at[idx].get(mode="promise_in_bounds")

        def init_body(i, _):
            cmin_v[pl.ds(i * _L, _L)] = inf16
            return 0
        lax.fori_loop(0, nkv, init_body, 0)

        nkb = nkv // 4

        def qv_body(qv, racc):
            qvx = qs_v[pl.ds(qv * _L, _L)]
            qvy = qs_v[pl.ds(qpw + qv * _L, _L)]
            qvz = qs_v[pl.ds(2 * qpw + qv * _L, _L)]

            def make_kb_body(l0):
                def kb_body(kb, rms):
                    base = kb * 4 * _L
                    kx = [keys_v[0, pl.ds(base + v * _L, _L)] for v in range(4)]
                    ky = [keys_v[1, pl.ds(base + v * _L, _L)] for v in range(4)]
                    kz = [keys_v[2, pl.ds(base + v * _L, _L)] for v in range(4)]
                    cm = [cmin_v[pl.ds(base + v * _L, _L)] for v in range(4)]
                    new_rms = []
                    for l in range(l0, l0 + 8):
                        qx = splat(qvx, l)
                        qy = splat(qvy, l)
                        qz = splat(qvz, l)
                        rm = rms[l - l0]
                        for v in range(4):
                            dx = qx - kx[v]
                            t = dx * dx
                            dy = qy - ky[v]
                            t = dy * dy + t
                            dz = qz - kz[v]
                            t = dz * dz + t
                            rm = jnp.minimum(rm, t)
                            cm[v] = jnp.minimum(cm[v], t)
                        new_rms.append(rm)
                    for v in range(4):
                        cmin_v[pl.ds(base + v * _L, _L)] = cm[v]
                    return tuple(new_rms)
                return kb_body

            for l0 in (0, 8):
                rms = lax.fori_loop(0, nkb, make_kb_body(l0), (inf16,) * 8)
                for l in range(8):
                    racc = racc + _sqrt16(lanemin(rms[l]))
            return racc

        racc = lax.fori_loop(0, qv_n, qv_body, jnp.zeros((_L,), jnp.float32))
        racc_v[...] = racc
        pltpu.sync_copy(racc_v, rowp_hbm.at[wid])

        pltpu.sync_copy(cmin_v, shared.at[s])
        plsc.subcore_barrier()

        @pl.when(s % g == 0)
        def _():
            for j in range(g):
                pltpu.sync_copy(shared.at[s + j], tmp_v.at[j])

            def c_body(i, acc):
                sl = pl.ds(i * _L, _L)
                m = tmp_v[0, sl]
                for j in range(1, g):
                    m = jnp.minimum(m, tmp_v[j, sl])
                return acc + _sqrt16(m)

            cacc = lax.fori_loop(0, nkv, c_body, jnp.zeros((_L,), jnp.float32))
            cacc_v[...] = cacc
            pltpu.sync_copy(cacc_v, colp_hbm.at[b])

    return sc_chamfer


_TM = 256


def _tc_chamfer_body(pc1t_ref, flowt_ref, pc2t_ref, out_ref):
    n = pc1t_ref.shape[2]
    p1t = pc1t_ref[0] + flowt_ref[0]
    p1 = jnp.transpose(p1t)
    c2x = pc2t_ref[0, 0:1, :]
    c2y = pc2t_ref[0, 1:2, :]
    c2z = pc2t_ref[0, 2:3, :]

    row_sum = jnp.zeros((), dtype=jnp.float32)
    col_min = jnp.full((1, n), jnp.inf, dtype=jnp.float32)
    for i in range(n // _TM):
        sl = p1[i * _TM:(i + 1) * _TM]
        dx = sl[:, 0:1] - c2x
        d = dx * dx
        dy = sl[:, 1:2] - c2y
        d += dy * dy
        dz = sl[:, 2:3] - c2z
        d += dz * dz
        row_min = jnp.min(d, axis=1)
        row_sum += jnp.sum(jnp.sqrt(row_min))
        col_min = jnp.minimum(col_min, jnp.min(d, axis=0, keepdims=True))

    total = row_sum + jnp.sum(jnp.sqrt(col_min))
    out_ref[0] = total.reshape(1, 1)


def _tc_chamfer(pc1t, pc2t, flowt):
    b, _, n = pc1t.shape
    return pl.pallas_call(
        _tc_chamfer_body,
        grid=(b,),
        in_specs=[
            pl.BlockSpec((1, 3, n), lambda i: (i, 0, 0)),
            pl.BlockSpec((1, 3, n), lambda i: (i, 0, 0)),
            pl.BlockSpec((1, 3, n), lambda i: (i, 0, 0)),
        ],
        out_specs=pl.BlockSpec((1, 1, 1), lambda i: (i, 0, 0)),
        out_shape=jax.ShapeDtypeStruct((b, 1, 1), jnp.float32),
        compiler_params=pltpu.CompilerParams(
            dimension_semantics=("parallel",),
        ),
    )(pc1t, flowt, pc2t)


_SCB = 2


def kernel(pc1, pc2, flow):
    b, n, _ = pc1.shape
    tcb = b - _SCB
    g = _NW // _SCB
    qpw = n // g
    pc2t = jnp.transpose(pc2, (0, 2, 1))
    pc1t = jnp.transpose(pc1, (0, 2, 1))
    flowt = jnp.transpose(flow, (0, 2, 1))

    tc_part = _tc_chamfer(pc1t, pc2t, flowt)
    return jnp.sum(tc_part) / (b * n)
